# Initial kernel scaffold; baseline (speedup 1.0000x reference)
import jax
import jax.numpy as jnp
from jax.experimental import pallas as pl


def kernel(labels, coord0, coord1, coord2, coord3, coord4, feat0, feat1, feat2, feat3, feat4, num_classes):
    raise NotImplementedError("write your pallas kernel here")


# speedup vs baseline: 16.5299x; 16.5299x over previous
import functools

import jax
import jax.numpy as jnp
from jax import lax
from jax.experimental import pallas as pl
from jax.experimental.pallas import tpu as pltpu

_NS = [64, 32, 16, 8, 4]


def _kth_smallest_bits(di, k):
    rows = di.shape[0]

    def body(t, lo):
        bit = jnp.left_shift(jnp.int32(1), 30 - t)
        mid = lo + bit
        c = jnp.sum((di < mid).astype(jnp.int32), axis=1, keepdims=True)
        return jnp.where(c < k, mid, lo)

    return lax.fori_loop(0, 31, body, jnp.zeros((rows, 1), jnp.int32))


def _sq_dist(cq, crt):
    d = -2.0 * jnp.dot(cq, crt, preferred_element_type=jnp.float32)
    d = d + jnp.sum(cq * cq, axis=1, keepdims=True)
    d = d + jnp.sum(crt * crt, axis=0, keepdims=True)
    return d


def _label_prop_kernel(cq_ref, crt_ref, oh_ref, out_ref, *, k):
    cq = cq_ref[0]
    crt = crt_ref[0]
    d = jnp.maximum(_sq_dist(cq, crt), 0.0)
    di = lax.bitcast_convert_type(d, jnp.int32)
    v = _kth_smallest_bits(di, k)
    m = (di <= v).astype(jnp.float32)
    counts = jnp.dot(m, oh_ref[0], preferred_element_type=jnp.float32)
    out_ref[0, 0, :] = jnp.argmax(counts, axis=1).astype(jnp.int32)


def _stage_kernel(cq_ref, crt_ref, f_ref, ft_ref, lrow_ref, lcol_ref, out_ref,
                  *, ns, rblk):
    b = pl.program_id(0)
    rb = pl.program_id(1)
    cq = cq_ref[0]
    crt = crt_ref[0]
    rows, cols = rblk, crt.shape[1]

    d = jnp.maximum(_sq_dist(cq, crt), 0.0)
    di = lax.bitcast_convert_type(d, jnp.int32)
    v = _kth_smallest_bits(di, ns)
    m = di <= v

    lrow = lrow_ref[0]
    lcol = lcol_ref[0]
    eq = lrow == lcol
    cnt = jnp.sum((m & eq).astype(jnp.int32), axis=1, keepdims=True)
    pm = ((cnt > 0) & (cnt < ns)).astype(jnp.float32)

    f = f_ref[0]
    ft = ft_ref[0]
    f2 = -2.0 * jnp.dot(f, ft, preferred_element_type=jnp.float32)
    f2 = f2 + jnp.sum(f * f, axis=1, keepdims=True)
    f2 = f2 + jnp.sum(ft * ft, axis=0, keepdims=True)
    colid = lax.broadcasted_iota(jnp.int32, (rows, cols), 1)
    rowid = lax.broadcasted_iota(jnp.int32, (rows, cols), 0) + rb * rblk
    f2 = jnp.where(colid == rowid, 0.0, jnp.maximum(f2, 0.0))
    fdist = jnp.sqrt(f2 + 1e-6)

    minf = jnp.min(jnp.where(m, fdist, jnp.inf), axis=1, keepdims=True)
    e = jnp.where(m, jnp.exp(minf - fdist), 0.0)
    pos = jnp.sum(pm * jnp.sum(jnp.where(eq, e, 0.0), axis=1, keepdims=True))
    neg = jnp.sum(pm * jnp.sum(e, axis=1, keepdims=True))
    anyc = jnp.sum(pm)

    @pl.when((b == 0) & (rb == 0))
    def _():
        out_ref[...] = jnp.zeros_like(out_ref)

    lane = lax.broadcasted_iota(jnp.int32, (1, 128), 1)
    vec = (jnp.where(lane == 0, pos, 0.0) + jnp.where(lane == 1, neg, 0.0)
           + jnp.where(lane == 2, anyc, 0.0))
    out_ref[...] += vec


def _run_label_prop(coord_q, coord0_t, ohp, k):
    batch, s, _ = coord_q.shape
    n = coord0_t.shape[2]
    rblk = min(s, 128)
    grid = (batch, s // rblk)
    return pl.pallas_call(
        functools.partial(_label_prop_kernel, k=k),
        grid=grid,
        in_specs=[
            pl.BlockSpec((1, rblk, 3), lambda b, r: (b, r, 0)),
            pl.BlockSpec((1, 3, n), lambda b, r: (b, 0, 0)),
            pl.BlockSpec((1, n, 128), lambda b, r: (b, 0, 0)),
        ],
        out_specs=pl.BlockSpec((1, 1, rblk), lambda b, r: (b, 0, r)),
        out_shape=jax.ShapeDtypeStruct((batch, 1, s), jnp.int32),
        compiler_params=pltpu.CompilerParams(
            dimension_semantics=("arbitrary", "arbitrary")),
    )(coord_q, coord0_t, ohp)


def _run_stage(coord, coord_t, feat, feat_t, lrow, lcol, ns):
    batch, s, _ = coord.shape
    c = feat.shape[2]
    rblk = min(s, 128)
    grid = (batch, s // rblk)
    out = pl.pallas_call(
        functools.partial(_stage_kernel, ns=ns, rblk=rblk),
        grid=grid,
        in_specs=[
            pl.BlockSpec((1, rblk, 3), lambda b, r: (b, r, 0)),
            pl.BlockSpec((1, 3, s), lambda b, r: (b, 0, 0)),
            pl.BlockSpec((1, rblk, c), lambda b, r: (b, r, 0)),
            pl.BlockSpec((1, c, s), lambda b, r: (b, 0, 0)),
            pl.BlockSpec((1, rblk, 1), lambda b, r: (b, r, 0)),
            pl.BlockSpec((1, 1, s), lambda b, r: (b, 0, 0)),
        ],
        out_specs=pl.BlockSpec((1, 128), lambda b, r: (0, 0)),
        out_shape=jax.ShapeDtypeStruct((1, 128), jnp.float32),
        compiler_params=pltpu.CompilerParams(
            dimension_semantics=("arbitrary", "arbitrary")),
    )(coord, coord_t, feat, feat_t, lrow, lcol)
    return out[0, 0], out[0, 1], out[0, 2]


def kernel(labels, coord0, coord1, coord2, coord3, coord4,
           feat0, feat1, feat2, feat3, feat4, num_classes):
    coords = [coord0, coord1, coord2, coord3, coord4]
    feats = [feat0, feat1, feat2, feat3, feat4]

    labels_oh = jax.nn.one_hot(labels, 21, dtype=jnp.float32)
    labels_oh = labels_oh * (jnp.arange(21) < num_classes).astype(jnp.float32)
    ohp = jnp.pad(labels_oh, ((0, 0), (0, 0), (0, 128 - 21)))
    coord0_t = jnp.swapaxes(coord0, 1, 2)

    l0 = jnp.where(labels < num_classes, labels, 0).astype(jnp.int32)
    lcol0 = l0[:, None, :]

    loss = jnp.asarray(0.0, dtype=jnp.float32)
    for i, (coord, feat) in enumerate(zip(coords, feats)):
        if i == 0:
            lcol = lcol0
        else:
            lcol = _run_label_prop(coord, coord0_t, ohp, 4 * i)
        lrow = jnp.swapaxes(lcol, 1, 2)
        coord_t = jnp.swapaxes(coord, 1, 2)
        feat_t = jnp.swapaxes(feat, 1, 2)
        pos, neg, anyc = _run_stage(coord, coord_t, feat, feat_t,
                                    lrow, lcol, _NS[i])
        term = -jnp.log(pos / (neg + 1e-6))
        loss = loss + jnp.where(anyc > 0, term, 0.0)
    return loss


# speedup vs baseline: 28.3953x; 1.7178x over previous
import functools

import jax
import jax.numpy as jnp
from jax import lax
from jax.experimental import pallas as pl
from jax.experimental.pallas import tpu as pltpu

_NS = [64, 32, 16, 8, 4]


def _topk_mask(d, k):
    rows = d.shape[0]
    rowmax = jnp.maximum(jnp.max(d, axis=1, keepdims=True), 1e-30)
    q = (d * (65535.0 / rowmax)).astype(jnp.int32)

    def body(t, lo):
        mid = lo + jnp.left_shift(jnp.int32(1), 15 - t)
        c = jnp.sum((q < mid).astype(jnp.int32), axis=1, keepdims=True)
        return jnp.where(c < k, mid, lo)

    v = lax.fori_loop(0, 16, body, jnp.zeros((rows, 1), jnp.int32))
    return q <= v


def _sq_dist(cq, crt):
    d = -2.0 * jnp.dot(cq, crt, preferred_element_type=jnp.float32)
    d = d + jnp.sum(cq * cq, axis=1, keepdims=True)
    d = d + jnp.sum(crt * crt, axis=0, keepdims=True)
    return d


def _label_prop_kernel(cq_ref, crt_ref, oh_ref, out_ref, *, k):
    d = jnp.maximum(_sq_dist(cq_ref[0], crt_ref[0]), 0.0)
    m = _topk_mask(d, k).astype(jnp.float32)
    counts = jnp.dot(m, oh_ref[0], preferred_element_type=jnp.float32)
    out_ref[0, 0, :] = jnp.argmax(counts, axis=1).astype(jnp.int32)


def _stage_kernel(cq_ref, crt_ref, f_ref, ft_ref, lrow_ref, lcol_ref, out_ref,
                  *, ns, rblk):
    rb = pl.program_id(1)
    cq = cq_ref[0]
    crt = crt_ref[0]
    rows, cols = rblk, crt.shape[1]

    d = jnp.maximum(_sq_dist(cq, crt), 0.0)
    m = _topk_mask(d, ns)

    lrow = lrow_ref[0]
    lcol = lcol_ref[0]
    eq = lrow == lcol
    cnt = jnp.sum((m & eq).astype(jnp.int32), axis=1, keepdims=True)
    pm = ((cnt > 0) & (cnt < ns)).astype(jnp.float32)

    f = f_ref[0]
    ft = ft_ref[0]
    f2 = -2.0 * jnp.dot(f, ft, preferred_element_type=jnp.float32)
    f2 = f2 + jnp.sum(f * f, axis=1, keepdims=True)
    f2 = f2 + jnp.sum(ft * ft, axis=0, keepdims=True)
    colid = lax.broadcasted_iota(jnp.int32, (rows, cols), 1)
    rowid = lax.broadcasted_iota(jnp.int32, (rows, cols), 0) + rb * rblk
    f2 = jnp.where(colid == rowid, 0.0, jnp.maximum(f2, 0.0))
    fdist = jnp.sqrt(f2 + 1e-6)

    minf = jnp.min(jnp.where(m, fdist, jnp.inf), axis=1, keepdims=True)
    e = jnp.where(m, jnp.exp(minf - fdist), 0.0)
    pos = jnp.sum(pm * jnp.sum(jnp.where(eq, e, 0.0), axis=1, keepdims=True))
    neg = jnp.sum(pm * jnp.sum(e, axis=1, keepdims=True))
    anyc = jnp.sum(pm)

    lane = lax.broadcasted_iota(jnp.int32, (1, 128), 1)
    out_ref[0, 0] = (jnp.where(lane == 0, pos, 0.0)
                     + jnp.where(lane == 1, neg, 0.0)
                     + jnp.where(lane == 2, anyc, 0.0))


def _run_label_prop(coord_q, coord0_t, ohp, k):
    batch, s, _ = coord_q.shape
    n = coord0_t.shape[2]
    rblk = min(s, 128)
    grid = (batch, s // rblk)
    return pl.pallas_call(
        functools.partial(_label_prop_kernel, k=k),
        grid=grid,
        in_specs=[
            pl.BlockSpec((1, rblk, 3), lambda b, r: (b, r, 0)),
            pl.BlockSpec((1, 3, n), lambda b, r: (b, 0, 0)),
            pl.BlockSpec((1, n, 128), lambda b, r: (b, 0, 0)),
        ],
        out_specs=pl.BlockSpec((1, 1, rblk), lambda b, r: (b, 0, r)),
        out_shape=jax.ShapeDtypeStruct((batch, 1, s), jnp.int32),
        compiler_params=pltpu.CompilerParams(
            dimension_semantics=("parallel", "parallel")),
    )(coord_q, coord0_t, ohp)


def _run_stage(coord, coord_t, feat, feat_t, lrow, lcol, ns):
    batch, s, _ = coord.shape
    c = feat.shape[2]
    rblk = min(s, 128)
    nrb = s // rblk
    grid = (batch, nrb)
    out = pl.pallas_call(
        functools.partial(_stage_kernel, ns=ns, rblk=rblk),
        grid=grid,
        in_specs=[
            pl.BlockSpec((1, rblk, 3), lambda b, r: (b, r, 0)),
            pl.BlockSpec((1, 3, s), lambda b, r: (b, 0, 0)),
            pl.BlockSpec((1, rblk, c), lambda b, r: (b, r, 0)),
            pl.BlockSpec((1, c, s), lambda b, r: (b, 0, 0)),
            pl.BlockSpec((1, rblk, 1), lambda b, r: (b, r, 0)),
            pl.BlockSpec((1, 1, s), lambda b, r: (b, 0, 0)),
        ],
        out_specs=pl.BlockSpec((1, 1, 1, 128), lambda b, r: (b, r, 0, 0)),
        out_shape=jax.ShapeDtypeStruct((batch, nrb, 1, 128), jnp.float32),
        compiler_params=pltpu.CompilerParams(
            dimension_semantics=("parallel", "parallel")),
    )(coord, coord_t, feat, feat_t, lrow, lcol)
    sums = jnp.sum(out, axis=(0, 1, 2))
    return sums[0], sums[1], sums[2]


def kernel(labels, coord0, coord1, coord2, coord3, coord4,
           feat0, feat1, feat2, feat3, feat4, num_classes):
    coords = [coord0, coord1, coord2, coord3, coord4]
    feats = [feat0, feat1, feat2, feat3, feat4]

    labels_oh = jax.nn.one_hot(labels, 21, dtype=jnp.float32)
    labels_oh = labels_oh * (jnp.arange(21) < num_classes).astype(jnp.float32)
    ohp = jnp.pad(labels_oh, ((0, 0), (0, 0), (0, 128 - 21)))
    coord0_t = jnp.swapaxes(coord0, 1, 2)

    l0 = jnp.where(labels < num_classes, labels, 0).astype(jnp.int32)
    lcol0 = l0[:, None, :]

    loss = jnp.asarray(0.0, dtype=jnp.float32)
    for i, (coord, feat) in enumerate(zip(coords, feats)):
        if i == 0:
            lcol = lcol0
        else:
            lcol = _run_label_prop(coord, coord0_t, ohp, 4 * i)
        lrow = jnp.swapaxes(lcol, 1, 2)
        coord_t = jnp.swapaxes(coord, 1, 2)
        feat_t = jnp.swapaxes(feat, 1, 2)
        pos, neg, anyc = _run_stage(coord, coord_t, feat, feat_t,
                                    lrow, lcol, _NS[i])
        term = -jnp.log(pos / (neg + 1e-6))
        loss = loss + jnp.where(anyc > 0, term, 0.0)
    return loss


# speedup vs baseline: 28.7119x; 1.0112x over previous
import functools

import jax
import jax.numpy as jnp
from jax import lax
from jax.experimental import pallas as pl
from jax.experimental.pallas import tpu as pltpu

_NS = [64, 32, 16, 8, 4]


def _topk_mask(d, k):
    rows = d.shape[0]
    rowmax = jnp.maximum(jnp.max(d, axis=1, keepdims=True), 1e-30)
    q = (d * (65535.0 / rowmax)).astype(jnp.int32)

    def body(t, lo):
        mid = lo + jnp.left_shift(jnp.int32(1), 15 - t)
        c = jnp.sum((q < mid).astype(jnp.int32), axis=1, keepdims=True)
        return jnp.where(c < k, mid, lo)

    v = lax.fori_loop(0, 16, body, jnp.zeros((rows, 1), jnp.int32))
    return q <= v


def _label_prop_kernel(cq_ref, crt_ref, oh_ref, out_ref, *, k):
    d = jnp.dot(cq_ref[0], crt_ref[0], preferred_element_type=jnp.float32)
    m = _topk_mask(d, k).astype(jnp.float32)
    counts = jnp.dot(m, oh_ref[0], preferred_element_type=jnp.float32)
    out_ref[0, 0, :] = jnp.argmax(counts, axis=1).astype(jnp.int32)


def _stage_kernel(cq_ref, crt_ref, f_ref, ft_ref, lrow_ref, lcol_ref, out_ref,
                  *, ns, rblk):
    rb = pl.program_id(1)
    rows = rblk
    cols = crt_ref.shape[2]

    d = jnp.dot(cq_ref[0], crt_ref[0], preferred_element_type=jnp.float32)
    m = _topk_mask(d, ns)
    mf = m.astype(jnp.float32)

    eq = lrow_ref[0] == lcol_ref[0]
    meqf = jnp.where(eq, mf, 0.0)
    cnt = jnp.sum(meqf, axis=1, keepdims=True)
    pm = ((cnt > 0.5) & (cnt < ns - 0.5)).astype(jnp.float32)

    f2 = jnp.dot(f_ref[0], ft_ref[0], preferred_element_type=jnp.float32)
    colid = lax.broadcasted_iota(jnp.int32, (rows, cols), 1)
    rowid = lax.broadcasted_iota(jnp.int32, (rows, cols), 0) + rb * rblk
    f2 = jnp.where(colid == rowid, 0.0, jnp.maximum(f2, 0.0))
    fdist = jnp.sqrt(f2 + 1e-6)

    minf = jnp.min(jnp.where(m, fdist, jnp.inf), axis=1, keepdims=True)
    expf = jnp.exp(minf - fdist)
    pos = jnp.sum(pm * jnp.sum(expf * meqf, axis=1, keepdims=True))
    neg = jnp.sum(pm * jnp.sum(expf * mf, axis=1, keepdims=True))
    anyc = jnp.sum(pm)

    lane = lax.broadcasted_iota(jnp.int32, (1, 128), 1)
    out_ref[0, 0] = (jnp.where(lane == 0, pos, 0.0)
                     + jnp.where(lane == 1, neg, 0.0)
                     + jnp.where(lane == 2, anyc, 0.0))


def _aug(x):
    n2 = jnp.sum(x * x, axis=2, keepdims=True)
    ones = jnp.ones_like(n2)
    q = jnp.concatenate([-2.0 * x, n2, ones], axis=2)
    r = jnp.concatenate([jnp.swapaxes(x, 1, 2), jnp.swapaxes(ones, 1, 2),
                         jnp.swapaxes(n2, 1, 2)], axis=1)
    return q, r


def _run_label_prop(caug_q, c0aug_r, ohp, k):
    batch, s, ck = caug_q.shape
    n = c0aug_r.shape[2]
    rblk = min(s, 128)
    grid = (batch, s // rblk)
    return pl.pallas_call(
        functools.partial(_label_prop_kernel, k=k),
        grid=grid,
        in_specs=[
            pl.BlockSpec((1, rblk, ck), lambda b, r: (b, r, 0)),
            pl.BlockSpec((1, ck, n), lambda b, r: (b, 0, 0)),
            pl.BlockSpec((1, n, 128), lambda b, r: (b, 0, 0)),
        ],
        out_specs=pl.BlockSpec((1, 1, rblk), lambda b, r: (b, 0, r)),
        out_shape=jax.ShapeDtypeStruct((batch, 1, s), jnp.int32),
        compiler_params=pltpu.CompilerParams(
            dimension_semantics=("parallel", "parallel")),
    )(caug_q, c0aug_r, ohp)


def _run_stage(caug_q, caug_r, faug_q, faug_r, lrow, lcol, ns):
    batch, s, ck = caug_q.shape
    fk = faug_q.shape[2]
    rblk = min(s, 128)
    nrb = s // rblk
    grid = (batch, nrb)
    out = pl.pallas_call(
        functools.partial(_stage_kernel, ns=ns, rblk=rblk),
        grid=grid,
        in_specs=[
            pl.BlockSpec((1, rblk, ck), lambda b, r: (b, r, 0)),
            pl.BlockSpec((1, ck, s), lambda b, r: (b, 0, 0)),
            pl.BlockSpec((1, rblk, fk), lambda b, r: (b, r, 0)),
            pl.BlockSpec((1, fk, s), lambda b, r: (b, 0, 0)),
            pl.BlockSpec((1, rblk, 1), lambda b, r: (b, r, 0)),
            pl.BlockSpec((1, 1, s), lambda b, r: (b, 0, 0)),
        ],
        out_specs=pl.BlockSpec((1, 1, 1, 128), lambda b, r: (b, r, 0, 0)),
        out_shape=jax.ShapeDtypeStruct((batch, nrb, 1, 128), jnp.float32),
        compiler_params=pltpu.CompilerParams(
            dimension_semantics=("parallel", "parallel")),
    )(caug_q, caug_r, faug_q, faug_r, lrow, lcol)
    sums = jnp.sum(out, axis=(0, 1, 2))
    return sums[0], sums[1], sums[2]


def kernel(labels, coord0, coord1, coord2, coord3, coord4,
           feat0, feat1, feat2, feat3, feat4, num_classes):
    coords = [coord0, coord1, coord2, coord3, coord4]
    feats = [feat0, feat1, feat2, feat3, feat4]

    labels_oh = jax.nn.one_hot(labels, 21, dtype=jnp.float32)
    labels_oh = labels_oh * (jnp.arange(21) < num_classes).astype(jnp.float32)
    ohp = jnp.pad(labels_oh, ((0, 0), (0, 0), (0, 128 - 21)))
    _, c0aug_r = _aug(coord0)

    l0 = jnp.where(labels < num_classes, labels, 0).astype(jnp.int32)
    lcol0 = l0[:, None, :]

    loss = jnp.asarray(0.0, dtype=jnp.float32)
    for i, (coord, feat) in enumerate(zip(coords, feats)):
        caug_q, caug_r = _aug(coord)
        faug_q, faug_r = _aug(feat)
        if i == 0:
            lcol = lcol0
        else:
            lcol = _run_label_prop(caug_q, c0aug_r, ohp, 4 * i)
        lrow = jnp.swapaxes(lcol, 1, 2)
        pos, neg, anyc = _run_stage(caug_q, caug_r, faug_q, faug_r,
                                    lrow, lcol, _NS[i])
        term = -jnp.log(pos / (neg + 1e-6))
        loss = loss + jnp.where(anyc > 0, term, 0.0)
    return loss
